# baseline (speedup 1.0000x reference)
import functools

import jax
import jax.numpy as jnp
from jax.experimental import pallas as pl
from jax.experimental.pallas import tpu as pltpu

B = 4
D = 768
N = 1000
VOCAB_SHAPE = 100000


def _body(idx_ref, emb_ref, ws_ref, bs_ref, wc_ref, bc_ref, out_ref):
    x = emb_ref[0]
    h = jnp.tanh(
        jax.lax.dot_general(x, ws_ref[...], (((1,), (0,)), ((), ())),
                            preferred_element_type=jnp.float32)
        + bs_ref[...]
    )
    y = x + h
    out_ref[0] = (
        jax.lax.dot_general(y, wc_ref[...], (((1,), (0,)), ((), ())),
                            preferred_element_type=jnp.float32)
        + bc_ref[...]
    )


@functools.partial(jax.jit, static_argnames=())
def kernel(tokens, embed_table, W_s, b_s, W_c, b_c):
    last = tokens[:, -1].astype(jnp.int32)
    bs2 = b_s.reshape(1, D)
    bc2 = b_c.reshape(1, N)
    emb3 = embed_table.reshape(VOCAB_SHAPE, 1, D)
    grid_spec = pltpu.PrefetchScalarGridSpec(
        num_scalar_prefetch=1,
        grid=(B,),
        in_specs=[
            pl.BlockSpec((1, 1, D), lambda b, idx: (idx[b], 0, 0)),
            pl.BlockSpec((D, D), lambda b, idx: (0, 0)),
            pl.BlockSpec((1, D), lambda b, idx: (0, 0)),
            pl.BlockSpec((D, N), lambda b, idx: (0, 0)),
            pl.BlockSpec((1, N), lambda b, idx: (0, 0)),
        ],
        out_specs=pl.BlockSpec((1, 1, N), lambda b, idx: (b, 0, 0)),
    )
    logits = pl.pallas_call(
        _body,
        grid_spec=grid_spec,
        out_shape=jax.ShapeDtypeStruct((B, 1, N), jnp.float32),
    )(last, emb3, W_s, bs2, W_c, bc2)
    return (logits.reshape(B, N), None)


# speedup vs baseline: 73.0826x; 73.0826x over previous
import jax
import jax.numpy as jnp
from jax.experimental import pallas as pl
from jax.experimental.pallas import tpu as pltpu

B = 4
D = 768
N = 1000


def _body(idx_ref, emb_hbm, ws_ref, bs_ref, wc_ref, bc_ref, out_ref,
          x_ref, sems):
    for i in range(B):
        pltpu.make_async_copy(
            emb_hbm.at[pl.ds(idx_ref[i], 1), :],
            x_ref.at[pl.ds(i, 1), :],
            sems.at[i],
        ).start()
    for i in range(B):
        pltpu.make_async_copy(
            emb_hbm.at[pl.ds(idx_ref[i], 1), :],
            x_ref.at[pl.ds(i, 1), :],
            sems.at[i],
        ).wait()
    x = x_ref[...]
    h = jnp.tanh(
        jax.lax.dot_general(x, ws_ref[...], (((1,), (0,)), ((), ())),
                            preferred_element_type=jnp.float32)
        + bs_ref[...]
    )
    y = x + h
    out_ref[...] = (
        jax.lax.dot_general(y, wc_ref[...], (((1,), (0,)), ((), ())),
                            preferred_element_type=jnp.float32)
        + bc_ref[...]
    )


def kernel(tokens, embed_table, W_s, b_s, W_c, b_c):
    last = tokens[:, -1].astype(jnp.int32)
    bs2 = b_s.reshape(1, D)
    bc2 = b_c.reshape(1, N)
    logits = pl.pallas_call(
        _body,
        in_specs=[
            pl.BlockSpec(memory_space=pltpu.SMEM),
            pl.BlockSpec(memory_space=pl.ANY),
            pl.BlockSpec((D, D), lambda: (0, 0)),
            pl.BlockSpec((1, D), lambda: (0, 0)),
            pl.BlockSpec((D, N), lambda: (0, 0)),
            pl.BlockSpec((1, N), lambda: (0, 0)),
        ],
        out_specs=pl.BlockSpec((B, N), lambda: (0, 0)),
        out_shape=jax.ShapeDtypeStruct((B, N), jnp.float32),
        scratch_shapes=[
            pltpu.VMEM((B, D), jnp.float32),
            pltpu.SemaphoreType.DMA((B,)),
        ],
    )(last, embed_table, W_s, bs2, W_c, bc2)
    return (logits, None)
